# Initial kernel scaffold; baseline (speedup 1.0000x reference)
import jax
import jax.numpy as jnp
from jax.experimental import pallas as pl


def kernel(images, Ks, RTs, zs, yaws, rolls, pitchs, post_RTs):
    raise NotImplementedError("write your pallas kernel here")


# speedup vs baseline: 2.4455x; 2.4455x over previous
import functools

import jax
import jax.numpy as jnp
from jax import lax
from jax.experimental import pallas as pl
from jax.experimental.pallas import tpu as pltpu
from jax.experimental.pallas import tpu_sc as plsc

_B = 2
_N = 6
_NIMG = _B * _N
_H = 224
_W = 224
_C = 16
_NPIX = _H * _W
_TH = 256
_TW = 256
_NPTS = _TH * _TW
_NC = 2
_NS = 16
_NWORK = _NC * _NS
_PTSW = _NPTS // _NWORK
_CHUNK = 512
_NCHUNK = _PTSW // _CHUNK
_NGRP = _CHUNK // 16


def _bf16r(v):
    u = lax.bitcast_convert_type(v, jnp.int32)
    r = (u + 0x7FFF + ((u >> 16) & 1)) & jnp.int32(-65536)
    return lax.bitcast_convert_type(r, jnp.float32)


def _ipm_body(tab_hbm, coef_hbm, gx_hbm, gy_hbm, out_hbm,
              coef_v, gx_v, gy_v, w_v, rows_v, mx_v, sem):
    wid = lax.axis_index("s") * _NC + lax.axis_index("c")
    pltpu.sync_copy(coef_hbm, coef_v)
    pltpu.sync_copy(gx_hbm, gx_v)
    pltpu.sync_copy(gy_hbm, gy_v)
    row_base = wid * (_PTSW // _TW)

    def outer(t, carry):
        b = t // (_NCHUNK * _N)
        chunk = (t // _N) % _NCHUNK
        n = t % _N
        i = b * _N + n
        img_off = i * _NPIX
        crow = row_base + chunk * 2
        pbase = wid * _PTSW + chunk * _CHUNK

        cA = coef_v[i, 0]
        ay0, ay1, ay2 = cA[0], cA[1], cA[2]
        ax0, ax1, ax2 = cA[3], cA[4], cA[5]
        az0 = cA[6] * cA[9]
        az1 = cA[7] * cA[9]
        az2 = cA[8] * cA[9]
        cB = coef_v[i, 1]

        def grp(gi, c2):
            r = crow + (gi >> 4)
            col0 = (gi & 15) * 16
            xs = gx_v[col0 >> 4]
            ys = gy_v[r]
            pb0 = _bf16r(ay0 * ys + ax0 * xs + az0)
            pb1 = _bf16r(ay1 * ys + ax1 * xs + az1)
            pb2 = _bf16r(ay2 * ys + ax2 * xs + az2)
            uv = cB[0] * pb0 + cB[1] * pb1 + cB[2] * pb2 + cB[3]
            vv = cB[4] * pb0 + cB[5] * pb1 + cB[6] * pb2 + cB[7]
            wv = cB[8] * pb0 + cB[9] * pb1 + cB[10] * pb2 + cB[11] + 1e-07
            px = uv / wv
            py = vv / wv
            pxs = jnp.minimum(jnp.maximum(px, -4.0), 228.0)
            pys = jnp.minimum(jnp.maximum(py, -4.0), 228.0)
            xt = pxs.astype(jnp.int32)
            yt = pys.astype(jnp.int32)
            x0 = jnp.where(xt.astype(jnp.float32) > pxs, xt - 1, xt)
            y0 = jnp.where(yt.astype(jnp.float32) > pys, yt - 1, yt)
            x1 = x0 + 1
            y1 = y0 + 1
            x0c = jnp.minimum(jnp.maximum(x0, 0), _W - 1)
            x1c = jnp.minimum(jnp.maximum(x1, 0), _W - 1)
            y0c = jnp.minimum(jnp.maximum(y0, 0), _H - 1)
            y1c = jnp.minimum(jnp.maximum(y1, 0), _H - 1)
            x0f = x0c.astype(jnp.float32)
            x1f = x1c.astype(jnp.float32)
            y0f = y0c.astype(jnp.float32)
            y1f = y1c.astype(jnp.float32)
            wx0 = x1f - px
            wx1 = px - x0f
            wy0 = y1f - py
            wy1 = py - y0f
            pos0 = gi * 16
            w_v[0, pl.ds(pos0, 16)] = wx0 * wy0
            w_v[1, pl.ds(pos0, 16)] = wx0 * wy1
            w_v[2, pl.ds(pos0, 16)] = wx1 * wy0
            w_v[3, pl.ds(pos0, 16)] = wx1 * wy1
            base0 = y0c * _W + img_off
            base1 = y1c * _W + img_off
            i00 = base0 + x0c
            i01 = base1 + x0c
            i10 = base0 + x1c
            i11 = base1 + x1c
            for s, iv in ((0, i00), (1, i01), (2, i10), (3, i11)):
                pltpu.async_copy(tab_hbm.at[iv],
                                 rows_v.at[pl.ds(s * _CHUNK + pos0, 16)], sem)
            return c2

        lax.fori_loop(0, _NGRP, grp, 0)
        pltpu.make_async_copy(tab_hbm.at[pl.ds(0, 4 * _CHUNK)], rows_v,
                              sem).wait()

        first = n == 0

        def blend(jb, c2):
            j0 = jb * 16
            w0 = w_v[0, pl.ds(j0, 16)]
            w1 = w_v[1, pl.ds(j0, 16)]
            w2 = w_v[2, pl.ds(j0, 16)]
            w3 = w_v[3, pl.ds(j0, 16)]
            for l in range(16):
                j = j0 + l
                acc = (w0[l] * rows_v[j]
                       + w1[l] * rows_v[_CHUNK + j]
                       + w2[l] * rows_v[2 * _CHUNK + j]
                       + w3[l] * rows_v[3 * _CHUNK + j])
                mx_v[j] = jnp.where(first, acc, jnp.maximum(mx_v[j], acc))
            return c2

        lax.fori_loop(0, _CHUNK // 16, blend, 0)

        @pl.when(n == _N - 1)
        def _():
            pltpu.sync_copy(mx_v, out_hbm.at[b, pl.ds(pbase, _CHUNK)])

        return carry

    lax.fori_loop(0, _B * _NCHUNK * _N, outer, 0)


_ipm_call = pl.kernel(
    _ipm_body,
    out_type=jax.ShapeDtypeStruct((_B, _NPTS, _C), jnp.float32),
    mesh=plsc.VectorSubcoreMesh(core_axis_name="c", subcore_axis_name="s",
                                num_cores=_NC, num_subcores=_NS),
    compiler_params=pltpu.CompilerParams(use_tc_tiling_on_sc=False),
    scratch_types=[
        pltpu.VMEM((_NIMG, 2, 16), jnp.float32),
        pltpu.VMEM((16, 16), jnp.float32),
        pltpu.VMEM((_TH, 16), jnp.float32),
        pltpu.VMEM((4, _CHUNK), jnp.float32),
        pltpu.VMEM((4 * _CHUNK, _C), jnp.float32),
        pltpu.VMEM((_CHUNK, _C), jnp.float32),
        pltpu.SemaphoreType.DMA,
    ],
)


def _rotation_from_euler(rolls, pitchs, yaws):
    si, sj, sk = (jnp.sin(jnp.deg2rad(rolls)), jnp.sin(jnp.deg2rad(pitchs)),
                  jnp.sin(jnp.deg2rad(yaws)))
    ci, cj, ck = (jnp.cos(jnp.deg2rad(rolls)), jnp.cos(jnp.deg2rad(pitchs)),
                  jnp.cos(jnp.deg2rad(yaws)))
    cc, cs = ci * ck, ci * sk
    sc, ss = si * ck, si * sk
    z = jnp.zeros_like(si)
    o = jnp.ones_like(si)
    row0 = jnp.stack([cj * ck, sj * sc - cs, sj * cc + ss, z], axis=-1)
    row1 = jnp.stack([cj * sk, sj * ss + cc, sj * cs - sc, z], axis=-1)
    row2 = jnp.stack([-sj, cj * si, cj * ci, z], axis=-1)
    row3 = jnp.stack([z, z, z, o], axis=-1)
    return jnp.stack([row0, row1, row2, row3], axis=1)


def _b16(a):
    u = lax.bitcast_convert_type(a, jnp.int32)
    r = (u + 0x7FFF + ((u >> 16) & 1)) & jnp.int32(-65536)
    return lax.bitcast_convert_type(r, jnp.float32)


@jax.jit
def kernel(images, Ks, RTs, zs, yaws, rolls, pitchs, post_RTs):
    Rb = _b16(_rotation_from_euler(rolls, pitchs, yaws))
    Pb = _b16((post_RTs @ (Ks @ RTs)).reshape(_NIMG, 4, 4))
    zb = _b16(zs)
    rep = jnp.arange(_NIMG) % _B
    ay = Rb[rep][:, 0:3, 0]
    ax = Rb[rep][:, 0:3, 1]
    rz = Rb[rep][:, 0:3, 2]
    cA = jnp.concatenate([ay, ax, rz, zb[rep][:, None],
                          jnp.zeros((_NIMG, 6), jnp.float32)], axis=1)
    cB = jnp.concatenate([Pb[:, 0:3, :].reshape(_NIMG, 12),
                          jnp.zeros((_NIMG, 4), jnp.float32)], axis=1)
    coef = jnp.stack([cA, cB], axis=1)
    grid = _b16(jnp.linspace(-32.0, 32.0, _TW, dtype=jnp.float32))
    gx = grid.reshape(16, 16)
    gy = jnp.tile(grid[:, None], (1, 16))
    tab = jnp.transpose(images, (0, 1, 3, 4, 2)).reshape(_NIMG * _NPIX, _C)
    out = _ipm_call(tab, coef, gx, gy)
    return jnp.transpose(out.reshape(_B, _TH, _TW, _C), (0, 3, 1, 2))
